# baseline (speedup 1.0000x reference)
import functools

import jax
import jax.numpy as jnp
from jax import lax
from jax.experimental import pallas as pl
from jax.experimental.pallas import tpu as pltpu
from jax.experimental.pallas import tpu_sc as plsc

_N = 16
_H = 4
_OMEGA0 = 30.0
_CIN = 64
_COUT = 64
_A = 16.0
_BT = 512
_NPB = 144
_NC = 2
_NS = 16
_NW = _NC * _NS
_SCC = 1024


_INV_PI = 0.31830988618367906
_PI_HI = 3.140625
_PI_MID = 9.676536e-4
_S1 = -0.1666666
_S2 = 0.008333097
_S3 = -0.00019812485
_S4 = 2.6129003e-06


def _fast_sin(z):
    kf = jnp.round(z * _INV_PI)
    k = kf.astype(jnp.int32)
    r = z - kf * _PI_HI
    r = r - kf * _PI_MID
    s = r * r
    p = _S4
    for c in (_S3, _S2, _S1, 1.0):
        p = p * s + c
    p = p * r
    return jnp.where((k & 1) == 1, -p, p)


def _rank_kernel(c_ref, tril_ref, tid_ref, rank_ref, hist_ref, base_ref):
    @pl.when(pl.program_id(0) == 0)
    def _():
        base_ref[...] = jnp.zeros_like(base_ref)

    cb = c_ref[...]
    affine = cb * _A
    xi = jnp.floor(affine[:, 0:1]).astype(jnp.int32) % _H
    yi = jnp.floor(affine[:, 1:2]).astype(jnp.int32) % _H
    tid = _H * xi + yi

    ids = lax.broadcasted_iota(jnp.int32, (1, _N), 1)
    onehot = jnp.where(tid == ids, 1.0, 0.0)
    cum = jnp.dot(tril_ref[...], onehot,
                  preferred_element_type=jnp.float32)
    cumg = cum + base_ref[...]
    rank = jnp.sum(jnp.where(tid == ids, cumg, 0.0),
                   axis=1, keepdims=True)

    tid_ref[...] = tid
    rank_ref[...] = rank.astype(jnp.int32)
    new_base = cumg[_BT - 1:_BT, :] + onehot[_BT - 1:_BT, :]
    base_ref[...] = new_base
    hist_ref[...] = new_base


def _pos_kernel(hist_ref, tid_ref, rank_ref, pos_ref, eblk_ref):
    cnt = hist_ref[...]
    pblk = jnp.floor((cnt + (_BT - 1.0)) / _BT)
    tj = lax.broadcasted_iota(jnp.int32, (_N, _N), 0)
    tt = lax.broadcasted_iota(jnp.int32, (_N, _N), 1)
    lt_incl = jnp.where(tj <= tt, 1.0, 0.0)
    cb = jnp.dot(pblk, lt_incl, preferred_element_type=jnp.float32)
    padoff = _BT * (cb - pblk)

    tid = tid_ref[...]
    ids = lax.broadcasted_iota(jnp.int32, (1, _N), 1)
    masked = jnp.where(tid == ids, padoff, 0.0)
    base = jnp.sum(masked, axis=1, keepdims=True)
    pos_ref[...] = base.astype(jnp.int32) + rank_ref[...]

    r0 = lax.broadcasted_iota(jnp.int32, (2, 128), 0)
    c0 = lax.broadcasted_iota(jnp.int32, (2, 128), 1)
    pb = (128 * r0 + c0).astype(jnp.float32)
    e = jnp.zeros((2, 128), dtype=jnp.int32)
    for t in range(_N):
        e = e + jnp.where(pb >= cb[0, t], 1, 0)
    eblk_ref[...] = jnp.minimum(e, _N - 1)


def _sc_scatter_body(x_hbm, pos_hbm, xs_hbm, pos_v, xrow_v, sem):
    wid = lax.axis_index("s") * _NC + lax.axis_index("c")
    tpw = 65536 // _NW
    for sc in range(tpw // _SCC):
        base = wid * tpw + sc * _SCC
        prow = pl.multiple_of(base // 128, 8)
        pltpu.sync_copy(pos_hbm.at[pl.ds(prow, _SCC // 128)], pos_v)
        for h in range(2):
            pltpu.sync_copy(x_hbm.at[pl.ds(base + h * _BT, _BT)], xrow_v)
            copies = []
            for j in range(_BT // 128):
                copies.append(pltpu.async_copy(
                    xrow_v.at[pl.ds(j * 128, 128)],
                    xs_hbm.at[pos_v.at[h * 4 + j]], sem))
            for c in copies:
                c.wait()


def _expert_matmul_kernel(eblk_ref, xs_ref, wt_ref, ys_ref):
    acc = jnp.dot(xs_ref[:, :_CIN], wt_ref[0],
                  preferred_element_type=jnp.float32)
    ys_ref[:, :_COUT] = _fast_sin(_OMEGA0 * acc)


def _sc_gather_body(ys_hbm, pos_hbm, out_hbm, pos_v, row_v, sem):
    wid = lax.axis_index("s") * _NC + lax.axis_index("c")
    tpw = 65536 // _NW
    for sc in range(tpw // _SCC):
        base = wid * tpw + sc * _SCC
        prow = pl.multiple_of(base // 128, 8)
        pltpu.sync_copy(pos_hbm.at[pl.ds(prow, _SCC // 128)], pos_v)
        for h in range(2):
            copies = []
            for j in range(_BT // 128):
                copies.append(pltpu.async_copy(
                    ys_hbm.at[pos_v.at[h * 4 + j]],
                    row_v.at[pl.ds(j * 128, 128)], sem))
            for c in copies:
                c.wait()
            pltpu.sync_copy(row_v, out_hbm.at[pl.ds(base + h * _BT, _BT)])


@jax.jit
def kernel(in_feats, in_coords, W):
    B = in_feats.shape[0]
    nblk = B // _BT
    coords = in_coords.reshape(B, 2)
    wt = jnp.transpose(W, (0, 2, 1))

    tril = jnp.tril(jnp.ones((_BT, _BT), jnp.float32), k=-1)

    tid2, rank2, hist = pl.pallas_call(
        _rank_kernel,
        grid=(nblk,),
        in_specs=[
            pl.BlockSpec((_BT, 2), lambda i: (i, 0)),
            pl.BlockSpec((_BT, _BT), lambda i: (0, 0)),
        ],
        out_specs=[
            pl.BlockSpec((_BT, 1), lambda i: (i, 0)),
            pl.BlockSpec((_BT, 1), lambda i: (i, 0)),
            pl.BlockSpec((1, _N), lambda i: (0, 0)),
        ],
        out_shape=[
            jax.ShapeDtypeStruct((B, 1), jnp.int32),
            jax.ShapeDtypeStruct((B, 1), jnp.int32),
            jax.ShapeDtypeStruct((1, _N), jnp.float32),
        ],
        scratch_shapes=[pltpu.VMEM((1, _N), jnp.float32)],
    )(coords, tril)

    pos2, eblk2 = pl.pallas_call(
        _pos_kernel,
        grid=(nblk,),
        in_specs=[
            pl.BlockSpec((1, _N), lambda i: (0, 0)),
            pl.BlockSpec((_BT, 1), lambda i: (i, 0)),
            pl.BlockSpec((_BT, 1), lambda i: (i, 0)),
        ],
        out_specs=[
            pl.BlockSpec((_BT, 1), lambda i: (i, 0)),
            pl.BlockSpec((2, 128), lambda i: (0, 0)),
        ],
        out_shape=[
            jax.ShapeDtypeStruct((B, 1), jnp.int32),
            jax.ShapeDtypeStruct((2, 128), jnp.int32),
        ],
    )(hist, tid2, rank2)

    pos = pos2.reshape(B // 128, 128)
    eblk = eblk2.reshape(256)[:_NPB]

    xpad = jnp.pad(in_feats, ((0, 0), (0, 128 - _CIN)))
    mesh = plsc.VectorSubcoreMesh(core_axis_name="c", subcore_axis_name="s")
    xs = pl.kernel(
        _sc_scatter_body,
        mesh=mesh,
        out_type=jax.ShapeDtypeStruct((_NPB * _BT, 128), jnp.float32),
        scratch_types=[
            pltpu.VMEM((_SCC // 128, 128), jnp.int32),
            pltpu.VMEM((_BT, 128), jnp.float32),
            pltpu.SemaphoreType.DMA,
        ],
    )(xpad, pos)

    ys = pl.pallas_call(
        _expert_matmul_kernel,
        grid_spec=pltpu.PrefetchScalarGridSpec(
            num_scalar_prefetch=1,
            grid=(_NPB,),
            in_specs=[
                pl.BlockSpec((_BT, 128), lambda pb, eref: (pb, 0)),
                pl.BlockSpec((1, _CIN, _COUT), lambda pb, eref: (eref[pb], 0, 0)),
            ],
            out_specs=pl.BlockSpec((_BT, 128), lambda pb, eref: (pb, 0)),
        ),
        out_shape=jax.ShapeDtypeStruct((_NPB * _BT, 128), jnp.float32),
    )(eblk, xs, wt)

    out = pl.kernel(
        _sc_gather_body,
        mesh=mesh,
        out_type=jax.ShapeDtypeStruct((B, 128), jnp.float32),
        scratch_types=[
            pltpu.VMEM((_SCC // 128, 128), jnp.int32),
            pltpu.VMEM((_BT, 128), jnp.float32),
            pltpu.SemaphoreType.DMA,
        ],
    )(ys, pos)

    return out[:, :_COUT]


# speedup vs baseline: 2.2841x; 2.2841x over previous
import jax
import jax.numpy as jnp
from jax import lax
from jax.experimental import pallas as pl

_N = 16
_H = 4
_OMEGA0 = 30.0
_CIN = 64
_COUT = 64
_A = 16.0

_INV_PI = 0.31830988618367906
_PI_HI = 3.140625
_PI_MID = 9.676536e-4
_PI_LO = 5.126566e-12
_S0 = 1.0
_S1 = -0.1666666
_S2 = 0.008333097
_S3 = -0.00019812485
_S4 = 2.6129003e-06


def _fast_sin(z):
    kf = jnp.round(z * _INV_PI)
    k = kf.astype(jnp.int32)
    r = z - kf * _PI_HI
    r = r - kf * _PI_MID
    s = r * r
    p = _S4
    for c in (_S3, _S2, _S1, _S0):
        p = p * s + c
    p = p * r
    return jnp.where((k & 1) == 1, -p, p)


def _moe_block_kernel(x_ref, c_ref, w_ref, e_ref, s_ref, o_ref):
    xb = x_ref[...]
    cb = c_ref[...]
    wcat = w_ref[...]
    exp_cols = e_ref[...]
    sel = s_ref[...]

    affine = cb * _A
    xi = jnp.floor(affine[:, 0:1]).astype(jnp.int32) % _H
    yi = jnp.floor(affine[:, 1:2]).astype(jnp.int32) % _H
    tid = _H * xi + yi

    y = jnp.dot(xb, wcat, preferred_element_type=jnp.float32)

    ids = lax.broadcasted_iota(jnp.int32, (1, _N), 1)
    onehot = jnp.where(tid == ids, 1.0, 0.0).astype(jnp.bfloat16)
    maskf = jnp.dot(onehot, exp_cols,
                    preferred_element_type=jnp.float32)
    z = y * maskf
    acc = jnp.dot(z, sel, preferred_element_type=jnp.float32)
    o_ref[...] = _fast_sin(_OMEGA0 * acc)


@jax.jit
def kernel(in_feats, in_coords, W):
    B = in_feats.shape[0]
    bt = 4096
    coords = in_coords.reshape(B, 2)
    wcat = jnp.transpose(W.reshape(_N * _COUT, _CIN))
    sel = jnp.tile(jnp.eye(_COUT, dtype=jnp.float32), (_N, 1))
    exp_cols = jnp.repeat(jnp.eye(_N, dtype=jnp.bfloat16), _COUT, axis=1)

    out = pl.pallas_call(
        _moe_block_kernel,
        grid=(B // bt,),
        in_specs=[
            pl.BlockSpec((bt, _CIN), lambda i: (i, 0)),
            pl.BlockSpec((bt, 2), lambda i: (i, 0)),
            pl.BlockSpec((_CIN, _N * _COUT), lambda i: (0, 0)),
            pl.BlockSpec((_N, _N * _COUT), lambda i: (0, 0)),
            pl.BlockSpec((_N * _COUT, _COUT), lambda i: (0, 0)),
        ],
        out_specs=pl.BlockSpec((bt, _COUT), lambda i: (i, 0)),
        out_shape=jax.ShapeDtypeStruct((B, _COUT), jnp.float32),
    )(in_feats, coords, wcat, exp_cols, sel)
    return out


# speedup vs baseline: 3.2676x; 1.4306x over previous
import jax
import jax.numpy as jnp
from jax import lax
from jax.experimental import pallas as pl

_N = 16
_H = 4
_OMEGA0 = 30.0
_CIN = 64
_COUT = 64
_A = 16.0

_INV_PI = 0.31830988618367906
_PI_HI = 3.140625
_PI_MID = 9.676536e-4
_PI_LO = 5.126566e-12
_S0 = 1.0
_S1 = -0.1666666
_S2 = 0.008333097
_S3 = -0.00019812485
_S4 = 2.6129003e-06


def _fast_sin(z):
    kf = jnp.round(z * _INV_PI)
    k = kf.astype(jnp.int32)
    r = z - kf * _PI_HI
    r = r - kf * _PI_MID
    s = r * r
    p = _S4
    for c in (_S3, _S2, _S1, _S0):
        p = p * s + c
    p = p * r
    return jnp.where((k & 1) == 1, -p, p)


def _moe_block_kernel(x_ref, c_ref, w_ref, s_ref, o_ref):
    xb = x_ref[...]
    cb = c_ref[...]
    wcat = w_ref[...]
    sel = s_ref[...]

    affine = cb * _A
    xi = jnp.floor(affine[:, 0]).astype(jnp.int32) % _H
    yi = jnp.floor(affine[:, 1]).astype(jnp.int32) % _H
    tid = _H * xi + yi

    y = jnp.dot(xb, wcat, preferred_element_type=jnp.float32)

    bt = xb.shape[0]
    col_expert = lax.broadcasted_iota(jnp.int32, (bt, _N * _COUT), 1) // _COUT
    mask = col_expert == tid[:, None]
    z = jnp.where(mask, y, 0.0)
    acc128 = z[:, 0:128]
    for g in range(1, _N // 2):
        acc128 = acc128 + z[:, g * 128:(g + 1) * 128]
    acc = jnp.dot(acc128, sel, preferred_element_type=jnp.float32)
    o_ref[...] = _fast_sin(_OMEGA0 * acc)


@jax.jit
def kernel(in_feats, in_coords, W):
    B = in_feats.shape[0]
    bt = 4096
    coords = in_coords.reshape(B, 2)
    wcat = jnp.transpose(W.reshape(_N * _COUT, _CIN))
    sel = jnp.tile(jnp.eye(_COUT, dtype=jnp.float32), (2, 1))

    out = pl.pallas_call(
        _moe_block_kernel,
        grid=(B // bt,),
        in_specs=[
            pl.BlockSpec((bt, _CIN), lambda i: (i, 0)),
            pl.BlockSpec((bt, 2), lambda i: (i, 0)),
            pl.BlockSpec((_CIN, _N * _COUT), lambda i: (0, 0)),
            pl.BlockSpec((2 * _COUT, _COUT), lambda i: (0, 0)),
        ],
        out_specs=pl.BlockSpec((bt, _COUT), lambda i: (i, 0)),
        out_shape=jax.ShapeDtypeStruct((B, _COUT), jnp.float32),
    )(in_feats, coords, wcat, sel)
    return out
